# baseline (speedup 1.0000x reference)
import functools
import jax
import jax.numpy as jnp
from jax import lax
from jax.experimental import pallas as pl
from jax.experimental.pallas import tpu as pltpu
from jax.experimental.pallas import tpu_sc as plsc

N_NODES = 10000
N_EDGES = 320000
D = 128

NC = 2
NS = 16
NW = NC * NS

CHUNK = 128
EDGES_PER_TILE = 10240
N_CHUNKS = EDGES_PER_TILE // CHUNK
N_HALVES = 2
HALF_CHUNKS = N_CHUNKS // N_HALVES
E_PAD = EDGES_PER_TILE * NW
ACC_ROWS = 10112
ROWS_PER_TILE = ACC_ROWS // NS


def _sc_body(x_hbm, src_hbm, dst_hbm, z_hbm, out_hbm,
             src_v, dst_v, rows_a, rows_b, acc, sem_a, sem_b):
    c = lax.axis_index("c")
    s = lax.axis_index("s")
    wid = s * NC + c

    pltpu.sync_copy(z_hbm, acc.at[pl.ds(s * ROWS_PER_TILE, ROWS_PER_TILE)])
    plsc.subcore_barrier()

    for h in range(N_HALVES):
        pltpu.sync_copy(src_hbm.at[wid, h], src_v)
        pltpu.sync_copy(dst_hbm.at[wid, h], dst_v)

        pltpu.async_copy(x_hbm.at[src_v.at[0]], rows_a, sem_a)

        def body(i, carry):
            ja = 2 * i
            jb = 2 * i + 1
            pltpu.async_copy(x_hbm.at[src_v.at[jb]], rows_b, sem_b)
            pltpu.make_async_copy(x_hbm.at[src_v.at[ja]], rows_a, sem_a).wait()
            pltpu.sync_copy(rows_a, acc.at[dst_v.at[ja]], add=True)
            jn = jnp.minimum(ja + 2, HALF_CHUNKS - 1)
            pltpu.async_copy(x_hbm.at[src_v.at[jn]], rows_a, sem_a)
            pltpu.make_async_copy(x_hbm.at[src_v.at[jb]], rows_b, sem_b).wait()
            pltpu.sync_copy(rows_b, acc.at[dst_v.at[jb]], add=True)
            return carry

        lax.fori_loop(0, HALF_CHUNKS // 2, body, 0)
        pltpu.make_async_copy(x_hbm.at[src_v.at[HALF_CHUNKS - 1]], rows_a,
                              sem_a).wait()

    plsc.subcore_barrier()
    pltpu.sync_copy(acc.at[pl.ds(s * ROWS_PER_TILE, ROWS_PER_TILE)],
                    out_hbm.at[c, pl.ds(s * ROWS_PER_TILE, ROWS_PER_TILE)])


_sc_ax = pl.kernel(
    _sc_body,
    out_type=jax.ShapeDtypeStruct((NC, ACC_ROWS, D), jnp.float32),
    mesh=plsc.VectorSubcoreMesh(core_axis_name="c", subcore_axis_name="s"),
    scratch_types=[
        pltpu.VMEM((HALF_CHUNKS, CHUNK), jnp.int32),
        pltpu.VMEM((HALF_CHUNKS, CHUNK), jnp.int32),
        pltpu.VMEM((CHUNK, D), jnp.float32),
        pltpu.VMEM((CHUNK, D), jnp.float32),
        pltpu.VMEM_SHARED((ACC_ROWS, D), jnp.float32),
        pltpu.SemaphoreType.DMA,
        pltpu.SemaphoreType.DMA,
    ],
)


ROW_BLK = 1000
N_BLKS = N_NODES // ROW_BLK


def _tc_body(p_ref, w_ref, b_ref, o_ref):
    s = p_ref[0] + p_ref[1]
    o_ref[...] = (
        jnp.dot(s, w_ref[...], preferred_element_type=jnp.float32) + b_ref[...]
    )


@jax.jit
def kernel(X, edge_index, W, b):
    src = edge_index[0].astype(jnp.int32)
    dst = edge_index[1].astype(jnp.int32)
    pad = E_PAD - N_EDGES
    src_p = jnp.concatenate([src, jnp.zeros((pad,), jnp.int32)])
    dst_p = jnp.concatenate([dst, jnp.full((pad,), N_NODES, jnp.int32)])
    src3 = src_p.reshape(NW, N_HALVES, HALF_CHUNKS, CHUNK)
    dst3 = dst_p.reshape(NW, N_HALVES, HALF_CHUNKS, CHUNK)
    zrows = jnp.zeros((ROWS_PER_TILE, D), jnp.float32)

    partials = _sc_ax(X, src3, dst3, zrows)

    out = pl.pallas_call(
        _tc_body,
        grid=(N_BLKS,),
        in_specs=[
            pl.BlockSpec((NC, ROW_BLK, D), lambda i: (0, i, 0)),
            pl.BlockSpec((D, D), lambda i: (0, 0)),
            pl.BlockSpec((1, D), lambda i: (0, 0)),
        ],
        out_specs=pl.BlockSpec((ROW_BLK, D), lambda i: (i, 0)),
        out_shape=jax.ShapeDtypeStruct((N_NODES, D), jnp.float32),
    )(partials, W, b.reshape(1, D))
    return out


# speedup vs baseline: 2.5825x; 2.5825x over previous
import functools
import jax
import jax.numpy as jnp
from jax import lax
from jax.experimental import pallas as pl
from jax.experimental.pallas import tpu as pltpu
from jax.experimental.pallas import tpu_sc as plsc

N_NODES = 10000
N_EDGES = 320000
D = 128

NC = 2
NS = 16
NW = NC * NS

CHUNK = 128
EDGES_PER_TILE = 10112
N_CHUNKS = EDGES_PER_TILE // CHUNK
E_PAD = EDGES_PER_TILE * NW
ACC_ROWS = 10112
ROWS_PER_TILE = ACC_ROWS // NS


def _sc_body(x_hbm, src_hbm, dst_hbm, z_hbm, out_hbm,
             src_v, dst_v, rows_v, acc, sem):
    c = lax.axis_index("c")
    s = lax.axis_index("s")
    wid = s * NC + c
    rs = pl.ds(s * ROWS_PER_TILE, ROWS_PER_TILE)

    pltpu.sync_copy(z_hbm, acc.at[rs])

    pltpu.sync_copy(src_hbm.at[wid], src_v)
    pltpu.sync_copy(dst_hbm.at[wid], dst_v)
    plsc.subcore_barrier()

    def body(j, carry):
        pltpu.async_copy(x_hbm.at[src_v.at[j]], rows_v, sem).wait()
        pltpu.sync_copy(rows_v, acc.at[dst_v.at[j]], add=True)
        return carry

    lax.fori_loop(0, N_CHUNKS, body, 0)

    plsc.subcore_barrier()
    pltpu.sync_copy(acc.at[rs], out_hbm.at[c, rs])


_sc_ax = pl.kernel(
    _sc_body,
    out_type=jax.ShapeDtypeStruct((NC, ACC_ROWS, D), jnp.float32),
    mesh=plsc.VectorSubcoreMesh(core_axis_name="c", subcore_axis_name="s"),
    scratch_types=[
        pltpu.VMEM((N_CHUNKS, CHUNK), jnp.int32),
        pltpu.VMEM((N_CHUNKS, CHUNK), jnp.int32),
        pltpu.VMEM((CHUNK, D), jnp.float32),
        pltpu.VMEM_SHARED((ACC_ROWS, D), jnp.float32),
        pltpu.SemaphoreType.DMA,
    ],
)


ROW_BLK = 1000
N_BLKS = N_NODES // ROW_BLK


def _tc_body(p_ref, w_ref, b_ref, o_ref):
    s = p_ref[0] + p_ref[1]
    o_ref[...] = (
        jnp.dot(s, w_ref[...], preferred_element_type=jnp.float32) + b_ref[...]
    )


@jax.jit
def kernel(X, edge_index, W, b):
    src = edge_index[0].astype(jnp.int32)
    dst = edge_index[1].astype(jnp.int32)
    pad = E_PAD - N_EDGES
    pad_dst = N_NODES + (jnp.arange(pad, dtype=jnp.int32) % (ACC_ROWS - N_NODES))
    pad_src = jnp.arange(pad, dtype=jnp.int32) % N_NODES
    src_p = jnp.concatenate([src, pad_src])
    dst_p = jnp.concatenate([dst, pad_dst])
    src3 = src_p.reshape(NW, N_CHUNKS, CHUNK)
    dst3 = dst_p.reshape(NW, N_CHUNKS, CHUNK)
    zrows = jnp.zeros((ROWS_PER_TILE, D), jnp.float32)

    partials = _sc_ax(X, src3, dst3, zrows)

    out = pl.pallas_call(
        _tc_body,
        grid=(N_BLKS,),
        in_specs=[
            pl.BlockSpec((NC, ROW_BLK, D), lambda i: (0, i, 0)),
            pl.BlockSpec((D, D), lambda i: (0, 0)),
            pl.BlockSpec((1, D), lambda i: (0, 0)),
        ],
        out_specs=pl.BlockSpec((ROW_BLK, D), lambda i: (i, 0)),
        out_shape=jax.ShapeDtypeStruct((N_NODES, D), jnp.float32),
    )(partials, W, b.reshape(1, D))
    return out


# speedup vs baseline: 2.5992x; 1.0065x over previous
import functools
import jax
import jax.numpy as jnp
from jax import lax
from jax.experimental import pallas as pl
from jax.experimental.pallas import tpu as pltpu
from jax.experimental.pallas import tpu_sc as plsc

N_NODES = 10000
N_EDGES = 320000
D = 128

NC = 2
NS = 16
NW = NC * NS

CHUNK = 128
EDGES_PER_TILE = 10112
N_CHUNKS = EDGES_PER_TILE // CHUNK
E_PAD = EDGES_PER_TILE * NW
ACC_ROWS = 10112
ROWS_PER_TILE = ACC_ROWS // NS


def _sc_body(x_hbm, src_hbm, dst_hbm, z_hbm, out_hbm,
             src_v, dst_v, rows_v, acc, sem, zsem):
    c = lax.axis_index("c")
    s = lax.axis_index("s")
    wid = s * NC + c
    rs = pl.ds(s * ROWS_PER_TILE, ROWS_PER_TILE)

    pltpu.async_copy(z_hbm, acc.at[rs], zsem)
    pltpu.sync_copy(src_hbm.at[wid], src_v)
    pltpu.sync_copy(dst_hbm.at[wid], dst_v)
    pltpu.make_async_copy(z_hbm, acc.at[rs], zsem).wait()
    plsc.subcore_barrier()

    def body(j, carry):
        pltpu.async_copy(x_hbm.at[src_v.at[j]], rows_v, sem).wait()
        pltpu.sync_copy(rows_v, acc.at[dst_v.at[j]], add=True)
        return carry

    lax.fori_loop(0, N_CHUNKS, body, 0)

    plsc.subcore_barrier()
    pltpu.sync_copy(acc.at[rs], out_hbm.at[c, rs])


_sc_ax = pl.kernel(
    _sc_body,
    out_type=jax.ShapeDtypeStruct((NC, ACC_ROWS, D), jnp.float32),
    mesh=plsc.VectorSubcoreMesh(core_axis_name="c", subcore_axis_name="s"),
    scratch_types=[
        pltpu.VMEM((N_CHUNKS, CHUNK), jnp.int32),
        pltpu.VMEM((N_CHUNKS, CHUNK), jnp.int32),
        pltpu.VMEM((CHUNK, D), jnp.float32),
        pltpu.VMEM_SHARED((ACC_ROWS, D), jnp.float32),
        pltpu.SemaphoreType.DMA,
        pltpu.SemaphoreType.DMA,
    ],
)


ROW_BLK = 1000
N_BLKS = N_NODES // ROW_BLK


def _tc_body(p_ref, w_ref, b_ref, o_ref):
    s = p_ref[0] + p_ref[1]
    o_ref[...] = (
        jnp.dot(s, w_ref[...], preferred_element_type=jnp.float32) + b_ref[...]
    )


@jax.jit
def kernel(X, edge_index, W, b):
    src = edge_index[0].astype(jnp.int32)
    dst = edge_index[1].astype(jnp.int32)
    pad = E_PAD - N_EDGES
    pad_dst = N_NODES + (jnp.arange(pad, dtype=jnp.int32) % (ACC_ROWS - N_NODES))
    pad_src = jnp.arange(pad, dtype=jnp.int32) % N_NODES
    src_p = jnp.concatenate([src, pad_src])
    dst_p = jnp.concatenate([dst, pad_dst])
    src3 = src_p.reshape(NW, N_CHUNKS, CHUNK)
    dst3 = dst_p.reshape(NW, N_CHUNKS, CHUNK)
    zrows = jnp.zeros((ROWS_PER_TILE, D), jnp.float32)

    partials = _sc_ax(X, src3, dst3, zrows)

    out = pl.pallas_call(
        _tc_body,
        grid=(N_BLKS,),
        in_specs=[
            pl.BlockSpec((NC, ROW_BLK, D), lambda i: (0, i, 0)),
            pl.BlockSpec((D, D), lambda i: (0, 0)),
            pl.BlockSpec((1, D), lambda i: (0, 0)),
        ],
        out_specs=pl.BlockSpec((ROW_BLK, D), lambda i: (i, 0)),
        out_shape=jax.ShapeDtypeStruct((N_NODES, D), jnp.float32),
    )(partials, W, b.reshape(1, D))
    return out


# speedup vs baseline: 2.9416x; 1.1318x over previous
import functools
import jax
import jax.numpy as jnp
from jax import lax
from jax.experimental import pallas as pl
from jax.experimental.pallas import tpu as pltpu
from jax.experimental.pallas import tpu_sc as plsc

N_NODES = 10000
N_EDGES = 320000
D = 128

NC = 2
NS = 16
NW = NC * NS

CHUNK = 128
EDGES_PER_TILE = 10240
N_CHUNKS = EDGES_PER_TILE // CHUNK
N_HALVES = 2
HALF_CHUNKS = N_CHUNKS // N_HALVES
E_PAD = EDGES_PER_TILE * NW
ACC_ROWS = 10112
ROWS_PER_TILE = ACC_ROWS // NS


def _sc_body(x_hbm, src_hbm, dst_hbm, z_hbm, out_hbm,
             src_v, dst_v, rows_a, rows_b, acc,
             gsem_a, gsem_b, ssem_a, ssem_b, zsem):
    c = lax.axis_index("c")
    s = lax.axis_index("s")
    wid = s * NC + c
    rs = pl.ds(s * ROWS_PER_TILE, ROWS_PER_TILE)

    pltpu.async_copy(z_hbm, acc.at[rs], zsem)
    pltpu.sync_copy(src_hbm.at[wid, 0], src_v)
    pltpu.sync_copy(dst_hbm.at[wid, 0], dst_v)
    pltpu.make_async_copy(z_hbm, acc.at[rs], zsem).wait()
    plsc.subcore_barrier()

    for h in range(N_HALVES):
        if h > 0:
            pltpu.sync_copy(src_hbm.at[wid, h], src_v)
            pltpu.sync_copy(dst_hbm.at[wid, h], dst_v)
        pltpu.async_copy(x_hbm.at[src_v.at[0]], rows_a, gsem_a)
        pltpu.async_copy(x_hbm.at[src_v.at[1]], rows_b, gsem_b)

        def body(i, carry):
            ja = 2 * i
            jb = 2 * i + 1
            pltpu.make_async_copy(x_hbm.at[src_v.at[ja]], rows_a, gsem_a).wait()
            pltpu.async_copy(rows_a, acc.at[dst_v.at[ja]], ssem_a, add=True)
            pltpu.make_async_copy(x_hbm.at[src_v.at[jb]], rows_b, gsem_b).wait()
            pltpu.async_copy(rows_b, acc.at[dst_v.at[jb]], ssem_b, add=True)
            jna = jnp.minimum(ja + 2, HALF_CHUNKS - 1)
            jnb = jnp.minimum(jb + 2, HALF_CHUNKS - 1)
            pltpu.make_async_copy(rows_a, acc.at[dst_v.at[ja]], ssem_a).wait()
            pltpu.async_copy(x_hbm.at[src_v.at[jna]], rows_a, gsem_a)
            pltpu.make_async_copy(rows_b, acc.at[dst_v.at[jb]], ssem_b).wait()
            pltpu.async_copy(x_hbm.at[src_v.at[jnb]], rows_b, gsem_b)
            return carry

        lax.fori_loop(0, HALF_CHUNKS // 2, body, 0)
        pltpu.make_async_copy(x_hbm.at[src_v.at[HALF_CHUNKS - 1]], rows_a,
                              gsem_a).wait()
        pltpu.make_async_copy(x_hbm.at[src_v.at[HALF_CHUNKS - 1]], rows_b,
                              gsem_b).wait()

    plsc.subcore_barrier()
    pltpu.sync_copy(acc.at[rs], out_hbm.at[c, rs])


_sc_ax = pl.kernel(
    _sc_body,
    out_type=jax.ShapeDtypeStruct((NC, ACC_ROWS, D), jnp.float32),
    mesh=plsc.VectorSubcoreMesh(core_axis_name="c", subcore_axis_name="s"),
    scratch_types=[
        pltpu.VMEM((HALF_CHUNKS, CHUNK), jnp.int32),
        pltpu.VMEM((HALF_CHUNKS, CHUNK), jnp.int32),
        pltpu.VMEM((CHUNK, D), jnp.float32),
        pltpu.VMEM((CHUNK, D), jnp.float32),
        pltpu.VMEM_SHARED((ACC_ROWS, D), jnp.float32),
        pltpu.SemaphoreType.DMA,
        pltpu.SemaphoreType.DMA,
        pltpu.SemaphoreType.DMA,
        pltpu.SemaphoreType.DMA,
        pltpu.SemaphoreType.DMA,
    ],
)


ROW_BLK = 1000
N_BLKS = N_NODES // ROW_BLK


def _tc_body(p_ref, w_ref, b_ref, o_ref):
    s = p_ref[0] + p_ref[1]
    o_ref[...] = (
        jnp.dot(s, w_ref[...], preferred_element_type=jnp.float32) + b_ref[...]
    )


@jax.jit
def kernel(X, edge_index, W, b):
    src = edge_index[0].astype(jnp.int32)
    dst = edge_index[1].astype(jnp.int32)
    pad = E_PAD - N_EDGES
    pad_dst = N_NODES + (jnp.arange(pad, dtype=jnp.int32) % (ACC_ROWS - N_NODES))
    pad_src = jnp.arange(pad, dtype=jnp.int32) % N_NODES
    src_p = jnp.concatenate([src, pad_src])
    dst_p = jnp.concatenate([dst, pad_dst])
    src3 = src_p.reshape(NW, N_HALVES, HALF_CHUNKS, CHUNK)
    dst3 = dst_p.reshape(NW, N_HALVES, HALF_CHUNKS, CHUNK)
    zrows = jnp.zeros((ROWS_PER_TILE, D), jnp.float32)

    partials = _sc_ax(X, src3, dst3, zrows)

    out = pl.pallas_call(
        _tc_body,
        grid=(N_BLKS,),
        in_specs=[
            pl.BlockSpec((NC, ROW_BLK, D), lambda i: (0, i, 0)),
            pl.BlockSpec((D, D), lambda i: (0, 0)),
            pl.BlockSpec((1, D), lambda i: (0, 0)),
        ],
        out_specs=pl.BlockSpec((ROW_BLK, D), lambda i: (i, 0)),
        out_shape=jax.ShapeDtypeStruct((N_NODES, D), jnp.float32),
    )(partials, W, b.reshape(1, D))
    return out


# speedup vs baseline: 3.2540x; 1.1062x over previous
import functools
import jax
import jax.numpy as jnp
from jax import lax
from jax.experimental import pallas as pl
from jax.experimental.pallas import tpu as pltpu
from jax.experimental.pallas import tpu_sc as plsc

N_NODES = 10000
N_EDGES = 320000
D = 128

NC = 2
NS = 16
NW = NC * NS

CHUNK = 64
NBUF = 4
EDGES_PER_TILE = 10240
N_CHUNKS = EDGES_PER_TILE // CHUNK
N_HALVES = 4
HALF_CHUNKS = N_CHUNKS // N_HALVES
E_PAD = EDGES_PER_TILE * NW
ACC_ROWS = 10112
ROWS_PER_TILE = ACC_ROWS // NS


def _sc_body(x_hbm, src_hbm, dst_hbm, z_hbm, out_hbm, *scr):
    src_v, dst_v = scr[0], scr[1]
    rows = list(scr[2:2 + NBUF])
    acc = scr[2 + NBUF]
    gsem = list(scr[3 + NBUF:3 + 2 * NBUF])
    ssem = list(scr[3 + 2 * NBUF:3 + 3 * NBUF])
    zsem = scr[3 + 3 * NBUF]

    c = lax.axis_index("c")
    s = lax.axis_index("s")
    wid = s * NC + c
    rs = pl.ds(s * ROWS_PER_TILE, ROWS_PER_TILE)

    pltpu.async_copy(z_hbm, acc.at[rs], zsem)
    pltpu.sync_copy(src_hbm.at[wid, 0], src_v)
    pltpu.sync_copy(dst_hbm.at[wid, 0], dst_v)
    pltpu.make_async_copy(z_hbm, acc.at[rs], zsem).wait()
    plsc.subcore_barrier()

    for h in range(N_HALVES):
        if h > 0:
            pltpu.sync_copy(src_hbm.at[wid, h], src_v)
            pltpu.sync_copy(dst_hbm.at[wid, h], dst_v)
        for b in range(NBUF):
            pltpu.async_copy(x_hbm.at[src_v.at[b]], rows[b], gsem[b])

        def body(i, carry):
            for b in range(NBUF):
                j = NBUF * i + b
                pltpu.make_async_copy(x_hbm.at[src_v.at[j]], rows[b],
                                      gsem[b]).wait()
                pltpu.async_copy(rows[b], acc.at[dst_v.at[j]], ssem[b],
                                 add=True)
            for b in range(NBUF):
                j = NBUF * i + b
                jn = jnp.minimum(j + NBUF, HALF_CHUNKS - 1)
                pltpu.make_async_copy(rows[b], acc.at[dst_v.at[j]],
                                      ssem[b]).wait()
                pltpu.async_copy(x_hbm.at[src_v.at[jn]], rows[b], gsem[b])
            return carry

        lax.fori_loop(0, HALF_CHUNKS // NBUF, body, 0)
        for b in range(NBUF):
            pltpu.make_async_copy(x_hbm.at[src_v.at[HALF_CHUNKS - 1]],
                                  rows[b], gsem[b]).wait()

    plsc.subcore_barrier()
    pltpu.sync_copy(acc.at[rs], out_hbm.at[c, rs])


_sc_ax = pl.kernel(
    _sc_body,
    out_type=jax.ShapeDtypeStruct((NC, ACC_ROWS, D), jnp.float32),
    mesh=plsc.VectorSubcoreMesh(core_axis_name="c", subcore_axis_name="s"),
    scratch_types=(
        [
            pltpu.VMEM((HALF_CHUNKS, CHUNK), jnp.int32),
            pltpu.VMEM((HALF_CHUNKS, CHUNK), jnp.int32),
        ]
        + [pltpu.VMEM((CHUNK, D), jnp.float32) for _ in range(NBUF)]
        + [pltpu.VMEM_SHARED((ACC_ROWS, D), jnp.float32)]
        + [pltpu.SemaphoreType.DMA for _ in range(2 * NBUF + 1)]
    ),
)


ROW_BLK = 1000
N_BLKS = N_NODES // ROW_BLK


def _tc_body(p_ref, w_ref, b_ref, o_ref):
    s = p_ref[0] + p_ref[1]
    o_ref[...] = (
        jnp.dot(s, w_ref[...], preferred_element_type=jnp.float32) + b_ref[...]
    )


@jax.jit
def kernel(X, edge_index, W, b):
    src = edge_index[0].astype(jnp.int32)
    dst = edge_index[1].astype(jnp.int32)
    pad = E_PAD - N_EDGES
    pad_dst = N_NODES + (jnp.arange(pad, dtype=jnp.int32) % (ACC_ROWS - N_NODES))
    pad_src = jnp.arange(pad, dtype=jnp.int32) % N_NODES
    src_p = jnp.concatenate([src, pad_src])
    dst_p = jnp.concatenate([dst, pad_dst])
    src3 = src_p.reshape(NW, N_HALVES, HALF_CHUNKS, CHUNK)
    dst3 = dst_p.reshape(NW, N_HALVES, HALF_CHUNKS, CHUNK)
    zrows = jnp.zeros((ROWS_PER_TILE, D), jnp.float32)

    partials = _sc_ax(X, src3, dst3, zrows)

    out = pl.pallas_call(
        _tc_body,
        grid=(N_BLKS,),
        in_specs=[
            pl.BlockSpec((NC, ROW_BLK, D), lambda i: (0, i, 0)),
            pl.BlockSpec((D, D), lambda i: (0, 0)),
            pl.BlockSpec((1, D), lambda i: (0, 0)),
        ],
        out_specs=pl.BlockSpec((ROW_BLK, D), lambda i: (i, 0)),
        out_shape=jax.ShapeDtypeStruct((N_NODES, D), jnp.float32),
    )(partials, W, b.reshape(1, D))
    return out
